# baseline (speedup 1.0000x reference)
import functools

import jax
import jax.numpy as jnp
from jax import lax
from jax.experimental import pallas as pl
from jax.experimental.pallas import tpu as pltpu
from jax.experimental.pallas import tpu_sc as plsc

_NC = 2
_NS = 16
_NW = _NC * _NS
_L = 16


@functools.lru_cache(maxsize=None)
def _make_transpose(D, V1):
  n_full = V1 // 128
  tail = V1 - n_full * 128
  blocks_per_w = (n_full + _NW - 1) // _NW
  mesh = plsc.VectorSubcoreMesh(core_axis_name="c", subcore_axis_name="s")

  @functools.partial(
      pl.kernel,
      mesh=mesh,
      compiler_params=pltpu.CompilerParams(needs_layout_passes=False),
      out_type=jax.ShapeDtypeStruct((V1, D), jnp.float32),
      scratch_types=[
          pltpu.VMEM((D, 128), jnp.float32),
          pltpu.VMEM((D, 128), jnp.float32),
          pltpu.VMEM((128, D), jnp.float32),
          pltpu.VMEM((128, D), jnp.float32),
          pltpu.SemaphoreType.DMA,
          pltpu.SemaphoreType.DMA,
      ],
  )
  def k(tablet_hbm, tail_hbm, scratch_hbm, blk0, blk1, trow0, trow1,
        sem_in, sem_out):
    blk = (blk0, blk1)
    trow = (trow0, trow1)
    wid = lax.axis_index("s") * _NC + lax.axis_index("c")

    rows16 = [lax.iota(jnp.int32, 16) + 16 * m for m in range(D // 16)]

    def valid(k_iter):
      c = wid * blocks_per_w + k_iter
      return (k_iter < blocks_per_w) & (c < n_full)

    def fetch(k_iter, slot):
      c = wid * blocks_per_w + k_iter

      @pl.when(valid(k_iter))
      def _():
        pltpu.async_copy(
            tablet_hbm.at[:, pl.ds(c * 128, 128)],
            blk[slot],
            sem_in,
        )

    def wait_fetch(k_iter, slot):
      @pl.when(valid(k_iter))
      def _():
        pltpu.make_async_copy(
            tablet_hbm.at[:, pl.ds(0, 128)], blk[slot], sem_in
        ).wait()

    def transpose_block(slot):
      for r0 in range(0, 128, 8):
        vals = []
        for r in range(r0, r0 + 8):
          cols = jnp.full((16,), r, jnp.int32)
          for m in range(D // 16):
            vals.append((r, m, plsc.load_gather(blk[slot], [rows16[m], cols])))
        for r, m, v in vals:
          trow[slot][r, pl.ds(m * 16, 16)] = v

    def store_block(k_iter, slot):
      c = wid * blocks_per_w + k_iter

      @pl.when(valid(k_iter))
      def _():
        transpose_block(slot)
        pltpu.async_copy(
            trow[slot],
            scratch_hbm.at[pl.ds(c * 128, 128), :],
            sem_out,
        )
        pltpu.make_async_copy(
            trow[slot], scratch_hbm.at[pl.ds(0, 128), :], sem_out
        ).wait()

    fetch(0, 0)

    @pl.loop(0, (blocks_per_w + 1) // 2)
    def _(h):
      k0 = h * 2
      wait_fetch(k0, 0)
      fetch(k0 + 1, 1)
      store_block(k0, 0)
      wait_fetch(k0 + 1, 1)
      fetch(k0 + 2, 0)
      store_block(k0 + 1, 1)

    if tail > 1:
      t = tail - 1

      @pl.when(wid == _NW - 1)
      def _():
        pltpu.sync_copy(tail_hbm, trow0.at[pl.ds(0, t), :])
        pltpu.sync_copy(
            trow0.at[pl.ds(0, t), :],
            scratch_hbm.at[pl.ds(n_full * 128, t), :],
        )

  return k


@functools.lru_cache(maxsize=None)
def _make_gather(V1, D, B):
  b_per_w = B // _NW
  mesh = plsc.VectorSubcoreMesh(core_axis_name="c", subcore_axis_name="s")

  @functools.partial(
      pl.kernel,
      mesh=mesh,
      out_type=jax.ShapeDtypeStruct((B, D), jnp.float32),
      scratch_types=[
          pltpu.VMEM((b_per_w,), jnp.int32),
          pltpu.VMEM((b_per_w, D), jnp.float32),
          pltpu.SemaphoreType.DMA,
      ],
  )
  def k(idx_hbm, table_hbm, out_hbm, idx_v, rows_v, sem):
    wid = lax.axis_index("s") * _NC + lax.axis_index("c")
    base = wid * b_per_w
    pltpu.sync_copy(idx_hbm.at[pl.ds(base, b_per_w)], idx_v)

    @pl.loop(0, b_per_w // 16)
    def _(g):
      vec = idx_v[pl.ds(g * 16, 16)]
      for j in range(16):
        pltpu.async_copy(
            table_hbm.at[pl.ds(vec[j], 1), :],
            rows_v.at[pl.ds(g * 16 + j, 1), :],
            sem,
        )

    pltpu.make_async_copy(
        table_hbm.at[pl.ds(0, b_per_w), :], rows_v, sem
    ).wait()
    pltpu.sync_copy(rows_v, out_hbm.at[pl.ds(base, b_per_w)])

  return k


def kernel(idx, table, p_uncond):
  B = idx.shape[0]
  V1, D = table.shape
  rkey = jax.random.fold_in(jax.random.key(0), 1)
  mask = jax.random.uniform(rkey, (B,)) < p_uncond
  idx = jnp.where(mask, V1 - 1, idx).astype(jnp.int32)
  n_full = V1 // 128
  tail_rows = lax.slice(table, (n_full * 128, 0), (V1 - 1, D))
  scratch = _make_transpose(D, V1)(table.T, tail_rows)
  out = _make_gather(V1, D, B)(idx, scratch)
  return out[:, None, :]


# speedup vs baseline: 1.0185x; 1.0185x over previous
import functools

import jax
import jax.numpy as jnp
from jax import lax
from jax.experimental import pallas as pl
from jax.experimental.pallas import tpu as pltpu
from jax.experimental.pallas import tpu_sc as plsc

_NC = 2
_NS = 16
_NW = _NC * _NS
_L = 16


@functools.lru_cache(maxsize=None)
def _make_transpose(D, V1):
  n_full = V1 // 128
  tail = V1 - n_full * 128
  blocks_per_w = (n_full + _NW - 1) // _NW
  mesh = plsc.VectorSubcoreMesh(core_axis_name="c", subcore_axis_name="s")

  @functools.partial(
      pl.kernel,
      mesh=mesh,
      compiler_params=pltpu.CompilerParams(needs_layout_passes=False),
      out_type=jax.ShapeDtypeStruct((V1, D), jnp.float32),
      scratch_types=[
          pltpu.VMEM((D, 129), jnp.float32),
          pltpu.VMEM((D, 129), jnp.float32),
          pltpu.VMEM((128, D), jnp.float32),
          pltpu.VMEM((128, D), jnp.float32),
          pltpu.SemaphoreType.DMA,
          pltpu.SemaphoreType.DMA,
      ],
  )
  def k(tablet_hbm, tail_hbm, scratch_hbm, blk0, blk1, trow0, trow1,
        sem_in, sem_out):
    blk = (blk0, blk1)
    trow = (trow0, trow1)
    wid = lax.axis_index("s") * _NC + lax.axis_index("c")

    rows16 = [lax.iota(jnp.int32, 16) + 16 * m for m in range(D // 16)]

    def valid(k_iter):
      c = wid * blocks_per_w + k_iter
      return (k_iter < blocks_per_w) & (c < n_full)

    def fetch(k_iter, slot):
      c = wid * blocks_per_w + k_iter

      @pl.when(valid(k_iter))
      def _():
        pltpu.async_copy(
            tablet_hbm.at[:, pl.ds(c * 128, 128)],
            blk[slot].at[:, pl.ds(0, 128)],
            sem_in,
        )

    def wait_fetch(k_iter, slot):
      @pl.when(valid(k_iter))
      def _():
        pltpu.make_async_copy(
            tablet_hbm.at[:, pl.ds(0, 128)],
            blk[slot].at[:, pl.ds(0, 128)],
            sem_in,
        ).wait()

    def transpose_block(slot):
      @pl.loop(0, 16)
      def _(g):
        r0 = g * 8
        vals = []
        for u in range(8):
          cols = jnp.full((16,), 0, jnp.int32) + (r0 + u)
          for m in range(D // 16):
            vals.append((u, m, plsc.load_gather(blk[slot], [rows16[m], cols])))
        for u, m, v in vals:
          trow[slot][r0 + u, pl.ds(m * 16, 16)] = v

    def store_block(k_iter, slot):
      c = wid * blocks_per_w + k_iter

      @pl.when(valid(k_iter))
      def _():
        transpose_block(slot)
        pltpu.async_copy(
            trow[slot],
            scratch_hbm.at[pl.ds(c * 128, 128), :],
            sem_out,
        )
        pltpu.make_async_copy(
            trow[slot], scratch_hbm.at[pl.ds(0, 128), :], sem_out
        ).wait()

    fetch(0, 0)

    @pl.loop(0, (blocks_per_w + 1) // 2)
    def _(h):
      k0 = h * 2
      wait_fetch(k0, 0)
      fetch(k0 + 1, 1)
      store_block(k0, 0)
      wait_fetch(k0 + 1, 1)
      fetch(k0 + 2, 0)
      store_block(k0 + 1, 1)

    if tail > 1:
      t = tail - 1

      @pl.when(wid == _NW - 1)
      def _():
        pltpu.sync_copy(tail_hbm, trow0.at[pl.ds(0, t), :])
        pltpu.sync_copy(
            trow0.at[pl.ds(0, t), :],
            scratch_hbm.at[pl.ds(n_full * 128, t), :],
        )

  return k


@functools.lru_cache(maxsize=None)
def _make_gather(V1, D, B):
  b_per_w = B // _NW
  mesh = plsc.VectorSubcoreMesh(core_axis_name="c", subcore_axis_name="s")

  @functools.partial(
      pl.kernel,
      mesh=mesh,
      out_type=jax.ShapeDtypeStruct((B, D), jnp.float32),
      scratch_types=[
          pltpu.VMEM((b_per_w,), jnp.int32),
          pltpu.VMEM((b_per_w, D), jnp.float32),
          pltpu.SemaphoreType.DMA,
      ],
  )
  def k(idx_hbm, table_hbm, out_hbm, idx_v, rows_v, sem):
    wid = lax.axis_index("s") * _NC + lax.axis_index("c")
    base = wid * b_per_w
    pltpu.sync_copy(idx_hbm.at[pl.ds(base, b_per_w)], idx_v)

    @pl.loop(0, b_per_w // 16)
    def _(g):
      vec = idx_v[pl.ds(g * 16, 16)]
      for j in range(16):
        pltpu.async_copy(
            table_hbm.at[pl.ds(vec[j], 1), :],
            rows_v.at[pl.ds(g * 16 + j, 1), :],
            sem,
        )

    pltpu.make_async_copy(
        table_hbm.at[pl.ds(0, b_per_w), :], rows_v, sem
    ).wait()
    pltpu.sync_copy(rows_v, out_hbm.at[pl.ds(base, b_per_w)])

  return k


def kernel(idx, table, p_uncond):
  B = idx.shape[0]
  V1, D = table.shape
  rkey = jax.random.fold_in(jax.random.key(0), 1)
  mask = jax.random.uniform(rkey, (B,)) < p_uncond
  idx = jnp.where(mask, V1 - 1, idx).astype(jnp.int32)
  n_full = V1 // 128
  tail_rows = lax.slice(table, (n_full * 128, 0), (V1 - 1, D))
  scratch = _make_transpose(D, V1)(table.T, tail_rows)
  out = _make_gather(V1, D, B)(idx, scratch)
  return out[:, None, :]


# speedup vs baseline: 2.5219x; 2.4760x over previous
import functools

import jax
import jax.numpy as jnp
from jax import lax
from jax.experimental import pallas as pl
from jax.experimental.pallas import tpu as pltpu
from jax.experimental.pallas import tpu_sc as plsc

_NC = 2
_NS = 16
_NW = _NC * _NS


@functools.lru_cache(maxsize=None)
def _make_gather(V1, D, B):
  b_per_w = B // _NW
  mesh = plsc.VectorSubcoreMesh(core_axis_name="c", subcore_axis_name="s")

  @functools.partial(
      pl.kernel,
      mesh=mesh,
      out_type=jax.ShapeDtypeStruct((B, D), jnp.float32),
      scratch_types=[
          pltpu.VMEM((b_per_w,), jnp.int32),
          pltpu.VMEM((b_per_w, D), jnp.float32),
          pltpu.SemaphoreType.DMA,
      ],
  )
  def k(idx_hbm, table_hbm, out_hbm, idx_v, rows_v, sem):
    wid = lax.axis_index("s") * _NC + lax.axis_index("c")
    base = wid * b_per_w
    pltpu.sync_copy(idx_hbm.at[pl.ds(base, b_per_w)], idx_v)

    @pl.loop(0, b_per_w // 16)
    def _(g):
      vec = idx_v[pl.ds(g * 16, 16)]
      for j in range(16):
        pltpu.async_copy(
            table_hbm.at[pl.ds(vec[j], 1), :],
            rows_v.at[pl.ds(g * 16 + j, 1), :],
            sem,
        )

    pltpu.make_async_copy(
        table_hbm.at[pl.ds(0, b_per_w), :], rows_v, sem
    ).wait()
    pltpu.sync_copy(rows_v, out_hbm.at[pl.ds(base, b_per_w)])

  return k


def kernel(idx, table, p_uncond):
  B = idx.shape[0]
  V1, D = table.shape
  rkey = jax.random.fold_in(jax.random.key(0), 1)
  mask = jax.random.uniform(rkey, (B,)) < p_uncond
  idx = jnp.where(mask, V1 - 1, idx).astype(jnp.int32)
  out = _make_gather(V1, D, B)(idx, table)
  return out[:, None, :]


# speedup vs baseline: 2.5239x; 1.0008x over previous
import functools

import jax
import jax.numpy as jnp
from jax import lax
from jax.experimental import pallas as pl
from jax.experimental.pallas import tpu as pltpu
from jax.experimental.pallas import tpu_sc as plsc

_NC = 2
_NS = 16
_NW = _NC * _NS


@functools.lru_cache(maxsize=None)
def _make_gather(V1, D, B):
  b_per_w = B // _NW
  mesh = plsc.VectorSubcoreMesh(core_axis_name="c", subcore_axis_name="s")

  @functools.partial(
      pl.kernel,
      mesh=mesh,
      compiler_params=pltpu.CompilerParams(skip_device_barrier=True),
      out_type=jax.ShapeDtypeStruct((B, D), jnp.float32),
      scratch_types=[
          pltpu.VMEM((b_per_w,), jnp.int32),
          pltpu.VMEM((b_per_w, D), jnp.float32),
          pltpu.SemaphoreType.DMA,
      ],
  )
  def k(idx_hbm, table_hbm, out_hbm, idx_v, rows_v, sem):
    wid = lax.axis_index("s") * _NC + lax.axis_index("c")
    base = wid * b_per_w
    pltpu.sync_copy(idx_hbm.at[pl.ds(base, b_per_w)], idx_v)

    @pl.loop(0, b_per_w // 16)
    def _(g):
      vec = idx_v[pl.ds(g * 16, 16)]
      for j in range(16):
        pltpu.async_copy(
            table_hbm.at[pl.ds(vec[j], 1), :],
            rows_v.at[pl.ds(g * 16 + j, 1), :],
            sem,
        )

    pltpu.make_async_copy(
        table_hbm.at[pl.ds(0, b_per_w), :], rows_v, sem
    ).wait()
    pltpu.sync_copy(rows_v, out_hbm.at[pl.ds(base, b_per_w)])

  return k


def kernel(idx, table, p_uncond):
  B = idx.shape[0]
  V1, D = table.shape
  rkey = jax.random.fold_in(jax.random.key(0), 1)
  mask = jax.random.uniform(rkey, (B,)) < p_uncond
  idx = jnp.where(mask, V1 - 1, idx).astype(jnp.int32)
  out = _make_gather(V1, D, B)(idx, table)
  return out[:, None, :]


# speedup vs baseline: 2.5312x; 1.0029x over previous
import functools

import jax
import jax.numpy as jnp
from jax import lax
from jax.experimental import pallas as pl
from jax.experimental.pallas import tpu as pltpu
from jax.experimental.pallas import tpu_sc as plsc

_NC = 2
_NS = 16
_NW = _NC * _NS


@functools.lru_cache(maxsize=None)
def _make_gather(V1, D, B):
  b_per_w = B // _NW
  mesh = plsc.VectorSubcoreMesh(core_axis_name="c", subcore_axis_name="s")

  @functools.partial(
      pl.kernel,
      mesh=mesh,
      out_type=jax.ShapeDtypeStruct((B, D), jnp.float32),
      scratch_types=[
          pltpu.VMEM((b_per_w,), jnp.int32),
          pltpu.VMEM((b_per_w, D), jnp.float32),
          pltpu.SemaphoreType.DMA,
      ],
  )
  def k(idx_hbm, table_hbm, out_hbm, idx_v, rows_v, sem):
    wid = lax.axis_index("s") * _NC + lax.axis_index("c")
    base = wid * b_per_w
    pltpu.sync_copy(idx_hbm.at[pl.ds(base, b_per_w)], idx_v)

    @pl.loop(0, b_per_w // 16)
    def _(g):
      vec = idx_v[pl.ds(g * 16, 16)]
      for j in range(16):
        pltpu.async_copy(
            table_hbm.at[pl.ds(vec[j], 1), :],
            rows_v.at[pl.ds(g * 16 + j, 1), :],
            sem,
        )

    pltpu.make_async_copy(
        table_hbm.at[pl.ds(0, b_per_w), :], rows_v, sem
    ).wait()
    pltpu.sync_copy(rows_v, out_hbm.at[pl.ds(base, b_per_w)])

  return k


def kernel(idx, table, p_uncond):
  B = idx.shape[0]
  V1, D = table.shape
  rkey = jax.random.fold_in(jax.random.key(0), 1)
  mask = jax.random.uniform(rkey, (B,)) < p_uncond
  idx = jnp.where(mask, V1 - 1, idx).astype(jnp.int32)
  out = _make_gather(V1, D, B)(idx, table)
  return out[:, None, :]
